# baseline (speedup 1.0000x reference)
import jax
import jax.numpy as jnp
from jax import lax
from jax.experimental import pallas as pl
from jax.experimental.pallas import tpu as pltpu
from jax.experimental.pallas import tpu_sc as plsc
import functools

_B = 4
_S = 2048
_HID = 768
_NC = 2
_NS = 16
_NW = _NC * _NS
_SPW = _S // _NW
_C = 16
_NG = _SPW // _C
_NRING = 2


def _make_kernel():
    mesh = plsc.VectorSubcoreMesh(core_axis_name="c", subcore_axis_name="s")

    @functools.partial(
        pl.kernel,
        mesh=mesh,
        out_type=jax.ShapeDtypeStruct((_B, _S, _HID), jnp.float32),
        scratch_types=(
            [pltpu.VMEM((_B, _SPW), jnp.int32)]
            + [pltpu.VMEM((_C, _HID), jnp.float32)
               for _ in range(_NRING * _B)]
            + [pltpu.VMEM((_C, _HID), jnp.float32)
               for _ in range(_NRING)]
            + [pltpu.SemaphoreType.DMA
               for _ in range(2 * _NRING * _B + _NRING)]
        ),
    )
    def emb_kernel(ids_hbm, word_hbm, pos_hbm, out_hbm, idx_v, *rest):
        nb = _NRING * _B
        wbufs = [rest[p * _B:(p + 1) * _B] for p in range(_NRING)]
        pbufs = rest[nb:nb + _NRING]
        gsems = rest[nb + _NRING:nb + _NRING + nb]
        ssems = rest[nb + _NRING + nb:nb + _NRING + 2 * nb]
        psems = rest[nb + _NRING + 2 * nb:nb + _NRING + 2 * nb + _NRING]

        wid = lax.axis_index("s") * _NC + lax.axis_index("c")
        s0 = wid * _SPW

        pdescs = [None] * _NG
        gdescs = [[None] * _B for _ in range(_NG)]
        sdescs = [[None] * _B for _ in range(_NG)]

        def start_pos(g):
            p = g % _NRING
            pdescs[g] = pltpu.async_copy(
                pos_hbm.at[pl.ds(s0 + g * _C, _C)], pbufs[p], psems[p])

        def start_gather(g, b):
            p = g % _NRING
            gdescs[g][b] = pltpu.async_copy(
                word_hbm.at[idx_v.at[b, pl.ds(g * _C, _C)]],
                wbufs[p][b], gsems[p * _B + b])

        def start_store(g, b):
            p = g % _NRING
            sdescs[g][b] = pltpu.async_copy(
                wbufs[p][b], out_hbm.at[b, pl.ds(s0 + g * _C, _C)],
                ssems[p * _B + b])

        start_pos(0)
        for b in range(_B):
            pltpu.sync_copy(ids_hbm.at[b, pl.ds(s0, _SPW)], idx_v.at[b])
            start_gather(0, b)

        def half_sweep(pbuf, wa, wb):
            @plsc.parallel_loop(0, _C)
            def add_row(i):
                for j in range(_HID // 16):
                    sl = pl.ds(j * 16, 16)
                    pv = pbuf[i, sl]
                    plsc.addupdate(wa.at[i, sl], pv)
                    plsc.addupdate(wb.at[i, sl], pv)

        for g in range(_NG):
            p = g % _NRING
            pdescs[g].wait()
            gdescs[g][0].wait()
            gdescs[g][1].wait()
            half_sweep(pbufs[p], wbufs[p][0], wbufs[p][1])
            start_store(g, 0)
            start_store(g, 1)
            if g + 1 < _NG:
                if g - 1 >= 0:
                    for b in range(_B):
                        sdescs[g - 1][b].wait()
                start_pos(g + 1)
                for b in range(_B):
                    start_gather(g + 1, b)
            gdescs[g][2].wait()
            gdescs[g][3].wait()
            half_sweep(pbufs[p], wbufs[p][2], wbufs[p][3])
            start_store(g, 2)
            start_store(g, 3)

        for g in range(_NG - 2, _NG):
            for b in range(_B):
                sdescs[g][b].wait()

    return emb_kernel


_emb_kernel = _make_kernel()


@jax.jit
def kernel(input_ids, word_embeddings, position_embeddings):
    ids = input_ids.astype(jnp.int32)
    return _emb_kernel(ids, word_embeddings, position_embeddings)


# speedup vs baseline: 1.0443x; 1.0443x over previous
import jax
import jax.numpy as jnp
from jax import lax
from jax.experimental import pallas as pl
from jax.experimental.pallas import tpu as pltpu
from jax.experimental.pallas import tpu_sc as plsc
import functools

_B = 4
_S = 2048
_HID = 768
_NC = 2
_NS = 16
_NW = _NC * _NS
_SPW = _S // _NW
_C = 32
_HPB = _SPW // _C
_NCHUNK = _B * _HPB
_NBUF = 3


def _make_kernel():
    mesh = plsc.VectorSubcoreMesh(core_axis_name="c", subcore_axis_name="s")

    @functools.partial(
        pl.kernel,
        mesh=mesh,
        out_type=jax.ShapeDtypeStruct((_B, _S, _HID), jnp.float32),
        scratch_types=(
            [pltpu.VMEM((_B, _SPW), jnp.int32),
             pltpu.VMEM((_SPW, _HID), jnp.float32)]
            + [pltpu.VMEM((_C, _HID), jnp.float32) for _ in range(_NBUF)]
            + [pltpu.SemaphoreType.DMA for _ in range(2 * _NBUF + 1)]
        ),
    )
    def emb_kernel(ids_hbm, word_hbm, pos_hbm, out_hbm, idx_v, pos_v, *rest):
        bufs = rest[:_NBUF]
        gsems = rest[_NBUF:2 * _NBUF]
        ssems = rest[2 * _NBUF:3 * _NBUF]
        psem = rest[3 * _NBUF]

        wid = lax.axis_index("s") * _NC + lax.axis_index("c")
        s0 = wid * _SPW

        gdescs = [None] * _NCHUNK
        sdescs = [None] * _NCHUNK

        def start_gather(c):
            b, h = c // _HPB, c % _HPB
            idx_ref = idx_v.at[b, pl.ds(h * _C, _C)]
            gdescs[c] = pltpu.async_copy(
                word_hbm.at[idx_ref], bufs[c % _NBUF], gsems[c % _NBUF])

        pos_desc = pltpu.async_copy(pos_hbm.at[pl.ds(s0, _SPW)], pos_v, psem)
        pltpu.sync_copy(ids_hbm.at[0, pl.ds(s0, _SPW)], idx_v.at[0])
        start_gather(0)
        for b in range(1, _B):
            pltpu.sync_copy(ids_hbm.at[b, pl.ds(s0, _SPW)], idx_v.at[b])
        pos_desc.wait()

        for c in range(_NCHUNK):
            if c + 1 < _NCHUNK:
                if c - 2 >= 0:
                    sdescs[c - 2].wait()
                start_gather(c + 1)
            b, h = c // _HPB, c % _HPB
            gdescs[c].wait()
            buf = bufs[c % _NBUF]

            @plsc.parallel_loop(0, _C)
            def add_row(i):
                for j in range(_HID // 16):
                    sl = pl.ds(j * 16, 16)
                    plsc.addupdate(buf.at[i, sl], pos_v[h * _C + i, sl])

            sdescs[c] = pltpu.async_copy(
                buf, out_hbm.at[b, pl.ds(s0 + h * _C, _C)], ssems[c % _NBUF])

        for c in range(_NCHUNK - 3, _NCHUNK):
            sdescs[c].wait()

    return emb_kernel


_emb_kernel = _make_kernel()


@jax.jit
def kernel(input_ids, word_embeddings, position_embeddings):
    ids = input_ids.astype(jnp.int32)
    return _emb_kernel(ids, word_embeddings, position_embeddings)
